# baseline (device time: 53718 ns/iter reference)
import jax
import jax.numpy as jnp
from jax import lax
from jax.experimental import pallas as pl
from jax.experimental.pallas import tpu as pltpu

N_DEV = 8
B = 2
SQL = 256
D = 512
HB = 4
DH = 64
SKV = 256

R_HOPS = 4
L_HOPS = 3

W_SIGMA = 0.02
QSCALE = 127.0 / (4.0 * W_SIGMA)

PF = 4


def _mm(a, b):
    return lax.dot_general(
        a, b, (((1,), (0,)), ((), ())), preferred_element_type=jnp.float32
    )


def _mm_t(a, b):
    return lax.dot_general(
        a, b, (((1,), (1,)), ((), ())), preferred_element_type=jnp.float32
    )


def kernel(x, Wq, K_ext, V_ext, Wo):
    def body(
        x_ref, wq_ref, k_ref, v_ref, wo_ref, out_ref,
        xb, kpf, vpf, rq_buf, ro_buf, lq_buf, lo_buf, pf_sem,
        rq_s, rq_r, ro_s, ro_r, lq_s, lq_r, lo_s, lo_r,
    ):
        my = lax.axis_index("i")
        left = (my - 1) % N_DEV
        right = (my + 1) % N_DEV

        def pf_start(slot, origin):
            copies = []
            for b in range(B):
                for hh in range(HB):
                    hd = origin * HB + hh
                    for src, dst in ((k_ref, kpf), (v_ref, vpf)):
                        copies.append(
                            pltpu.make_async_copy(
                                src.at[b, :, hd, :],
                                dst.at[slot, b, hh],
                                pf_sem.at[slot],
                            )
                        )
            for c in copies:
                c.start()
            return copies

        def pf_wait(copies):
            for c in copies:
                c.wait()

        order = [my]
        for d in range(1, R_HOPS + 1):
            order.append((my - d) % N_DEV)
            if d <= L_HOPS:
                order.append((my + d) % N_DEV)

        pf_handles = [pf_start(0, order[0]), pf_start(1, order[1]), None, None]

        barrier_sem = pltpu.get_barrier_semaphore()
        for nbr in (left, right):
            pl.semaphore_signal(
                barrier_sem, inc=1, device_id=(nbr,),
                device_id_type=pl.DeviceIdType.MESH,
            )
        pl.semaphore_wait(barrier_sem, 2)

        xb[...] = (
            x_ref[...].reshape(B * SQL, D) * (0.125 / QSCALE)
        ).astype(jnp.bfloat16)
        wq8 = jnp.clip(
            jnp.round(wq_ref[...] * QSCALE), -127.0, 127.0
        ).astype(jnp.int8)
        wo8 = jnp.clip(
            jnp.round(wo_ref[...] * QSCALE), -127.0, 127.0
        ).astype(jnp.int8)
        rq_buf[0] = wq8
        ro_buf[0] = wo8
        lq_buf[0] = wq8
        lo_buf[0] = wo8

        qi = lax.broadcasted_iota(jnp.int32, (SQL, SKV), 0)
        kj = lax.broadcasted_iota(jnp.int32, (SQL, SKV), 1)
        qb = my * HB + qi // 64
        kb = kj // 64
        mask = (qb == kb) | (kb == 0) | ((qb + kb) % 3 == 0)

        def contrib(qbuf, obuf, slot, pf_slot, first):
            wq_s = qbuf[slot].astype(jnp.bfloat16)
            wo_s = obuf[slot].astype(jnp.bfloat16)
            q16 = _mm(xb[...], wq_s).astype(jnp.bfloat16)
            parts = []
            for b in range(B):
                kblk = kpf[pf_slot, b].astype(jnp.bfloat16)
                vblk = vpf[pf_slot, b].astype(jnp.bfloat16)
                ctxs = []
                for h in range(HB):
                    qh = q16[b * SQL:(b + 1) * SQL, h * DH:(h + 1) * DH]
                    s = _mm_t(qh, kblk[h])
                    w = jnp.where(mask, jnp.exp(s), 0.0)
                    wsum = jnp.sum(w, axis=1, keepdims=True)
                    ctx = _mm(w.astype(jnp.bfloat16), vblk[h]) / wsum
                    ctxs.append(ctx.astype(jnp.bfloat16))
                parts.append(jnp.concatenate(ctxs, axis=1))
            ctx_all = jnp.concatenate(parts, axis=0)
            pall = _mm(ctx_all, wo_s).reshape(B, SQL, D)
            if first:
                out_ref[...] = pall
            else:
                out_ref[...] = out_ref[...] + pall

        def hop(qbuf, obuf, q_s, q_r, o_s, o_r, idx, dst):
            rd_q = pltpu.make_async_remote_copy(
                src_ref=qbuf.at[idx], dst_ref=qbuf.at[idx + 1],
                send_sem=q_s.at[idx], recv_sem=q_r.at[idx],
                device_id=(dst,), device_id_type=pl.DeviceIdType.MESH,
            )
            rd_o = pltpu.make_async_remote_copy(
                src_ref=obuf.at[idx], dst_ref=obuf.at[idx + 1],
                send_sem=o_s.at[idx], recv_sem=o_r.at[idx],
                device_id=(dst,), device_id_type=pl.DeviceIdType.MESH,
            )
            rd_q.start()
            rd_o.start()
            return rd_q, rd_o

        n_blk = 2

        def pf_next(k):
            nonlocal n_blk
            hi = min(2 * k + 2, N_DEV - 1)
            while n_blk <= hi:
                pf_handles[n_blk % PF] = pf_start(n_blk % PF, order[n_blk])
                n_blk += 1

        blk = 0
        for k in range(R_HOPS):
            r_rd = hop(rq_buf, ro_buf, rq_s, rq_r, ro_s, ro_r, k, right)
            l_rd = hop(lq_buf, lo_buf, lq_s, lq_r, lo_s, lo_r, k, left) \
                if k < L_HOPS else None
            pf_next(k)
            if k == 0:
                pf_wait(pf_handles[0])
                contrib(rq_buf, ro_buf, 0, 0, first=True)
                blk = 1
            else:
                pf_wait(pf_handles[blk % PF])
                contrib(rq_buf, ro_buf, k, blk % PF, first=False)
                blk += 1
                pf_wait(pf_handles[blk % PF])
                contrib(lq_buf, lo_buf, k, blk % PF, first=False)
                blk += 1
            for rd in r_rd:
                rd.wait()
            if l_rd is not None:
                for rd in l_rd:
                    rd.wait()
        pf_wait(pf_handles[blk % PF])
        contrib(rq_buf, ro_buf, R_HOPS, blk % PF, first=False)
        out_ref[...] = out_ref[...] * (1.0 / QSCALE)

    bf = jnp.bfloat16
    i8 = jnp.int8
    f32 = jnp.float32
    vmem = pl.BlockSpec(memory_space=pltpu.VMEM)
    hbm = pl.BlockSpec(memory_space=pltpu.MemorySpace.HBM)
    return pl.pallas_call(
        body,
        out_shape=jax.ShapeDtypeStruct((B, SQL, D), jnp.float32),
        in_specs=[vmem, vmem, hbm, hbm, vmem],
        out_specs=vmem,
        scratch_shapes=[
            pltpu.VMEM((B * SQL, D), bf),
            pltpu.VMEM((PF, B, HB, SKV, DH), f32),
            pltpu.VMEM((PF, B, HB, SKV, DH), f32),
            pltpu.VMEM((R_HOPS + 1, D, HB * DH), i8),
            pltpu.VMEM((R_HOPS + 1, HB * DH, D), i8),
            pltpu.VMEM((L_HOPS + 1, D, HB * DH), i8),
            pltpu.VMEM((L_HOPS + 1, HB * DH, D), i8),
            pltpu.SemaphoreType.DMA((PF,)),
            pltpu.SemaphoreType.DMA((R_HOPS,)),
            pltpu.SemaphoreType.DMA((R_HOPS,)),
            pltpu.SemaphoreType.DMA((R_HOPS,)),
            pltpu.SemaphoreType.DMA((R_HOPS,)),
            pltpu.SemaphoreType.DMA((L_HOPS,)),
            pltpu.SemaphoreType.DMA((L_HOPS,)),
            pltpu.SemaphoreType.DMA((L_HOPS,)),
            pltpu.SemaphoreType.DMA((L_HOPS,)),
        ],
        compiler_params=pltpu.CompilerParams(collective_id=0),
    )(x, Wq, K_ext, V_ext, Wo)


# device time: 35187 ns/iter; 1.5266x vs baseline; 1.5266x over previous
import jax
import jax.numpy as jnp
from jax import lax
from jax.experimental import pallas as pl
from jax.experimental.pallas import tpu as pltpu

N_DEV = 8
B = 2
SQL = 256
D = 512
HB = 4
DH = 64
SKV = 256

R_HOPS = 4
L_HOPS = 3

W_SIGMA = 0.02
QSCALE = 127.0 / (4.0 * W_SIGMA)


def _mm(a, b, out_dtype=jnp.float32):
    return lax.dot_general(
        a, b, (((1,), (0,)), ((), ())), preferred_element_type=out_dtype
    )


def kernel(x, Wq, K_ext, V_ext, Wo):
    K_r = jnp.transpose(K_ext, (0, 2, 3, 1)).astype(jnp.bfloat16)
    V_r = (jnp.transpose(V_ext, (0, 2, 1, 3)) * (1.0 / QSCALE)).astype(
        jnp.bfloat16
    )

    def body(
        x_ref, wq_ref, k_ref, v_ref, wo_ref, out_ref,
        xb, rq_buf, ro_buf, lq_buf, lo_buf,
        rq_s, rq_r, ro_s, ro_r, lq_s, lq_r, lo_s, lo_r,
    ):
        my = lax.axis_index("i")
        left = (my - 1) % N_DEV
        right = (my + 1) % N_DEV

        barrier_sem = pltpu.get_barrier_semaphore()
        for nbr in (left, right):
            pl.semaphore_signal(
                barrier_sem, inc=1, device_id=(nbr,),
                device_id_type=pl.DeviceIdType.MESH,
            )
        pl.semaphore_wait(barrier_sem, 2)

        xb[...] = (
            x_ref[...].reshape(B * SQL, D) * (0.125 / QSCALE)
        ).astype(jnp.bfloat16)
        wq8 = jnp.clip(
            jnp.round(wq_ref[...] * QSCALE), -127.0, 127.0
        ).astype(jnp.int8)
        wo8 = jnp.clip(
            jnp.round(wo_ref[...] * QSCALE), -127.0, 127.0
        ).astype(jnp.int8)
        rq_buf[0] = wq8
        ro_buf[0] = wo8
        lq_buf[0] = wq8
        lo_buf[0] = wo8

        qi = lax.broadcasted_iota(jnp.int32, (SQL, SKV), 0)
        kj = lax.broadcasted_iota(jnp.int32, (SQL, SKV), 1)
        qb = my * HB + qi // 64
        kb = kj // 64
        mask = (qb == kb) | (kb == 0) | ((qb + kb) % 3 == 0)

        def contrib(qbuf, obuf, slot, origin, first):
            wq_s = qbuf[slot].astype(jnp.bfloat16)
            wo_s = obuf[slot].astype(jnp.bfloat16)
            q16 = _mm(xb[...], wq_s).astype(jnp.bfloat16)
            parts = []
            for b in range(B):
                kblk = k_ref[b, pl.ds(origin * HB, HB)]
                vblk = v_ref[b, pl.ds(origin * HB, HB)]
                ctxs = []
                for h in range(HB):
                    qh = q16[b * SQL:(b + 1) * SQL, h * DH:(h + 1) * DH]
                    s = _mm(qh, kblk[h])
                    w = jnp.where(mask, jnp.exp(s), 0.0)
                    wsum = jnp.sum(w, axis=1, keepdims=True)
                    ctx = _mm(w.astype(jnp.bfloat16), vblk[h]) / wsum
                    ctxs.append(ctx.astype(jnp.bfloat16))
                parts.append(jnp.concatenate(ctxs, axis=1))
            ctx_all = jnp.concatenate(parts, axis=0)
            pall = _mm(ctx_all, wo_s).reshape(B, SQL, D)
            if first:
                out_ref[...] = pall
            else:
                out_ref[...] = out_ref[...] + pall

        def hop(qbuf, obuf, q_s, q_r, o_s, o_r, idx, dst):
            rd_q = pltpu.make_async_remote_copy(
                src_ref=qbuf.at[idx], dst_ref=qbuf.at[idx + 1],
                send_sem=q_s.at[idx], recv_sem=q_r.at[idx],
                device_id=(dst,), device_id_type=pl.DeviceIdType.MESH,
            )
            rd_o = pltpu.make_async_remote_copy(
                src_ref=obuf.at[idx], dst_ref=obuf.at[idx + 1],
                send_sem=o_s.at[idx], recv_sem=o_r.at[idx],
                device_id=(dst,), device_id_type=pl.DeviceIdType.MESH,
            )
            rd_q.start()
            rd_o.start()
            return rd_q, rd_o

        for k in range(R_HOPS):
            r_rd = hop(rq_buf, ro_buf, rq_s, rq_r, ro_s, ro_r, k, right)
            l_rd = hop(lq_buf, lo_buf, lq_s, lq_r, lo_s, lo_r, k, left) \
                if k < L_HOPS else None
            if k == 0:
                contrib(rq_buf, ro_buf, 0, my, first=True)
            else:
                contrib(rq_buf, ro_buf, k, (my - k) % N_DEV, first=False)
                contrib(lq_buf, lo_buf, k, (my + k) % N_DEV, first=False)
            for rd in r_rd:
                rd.wait()
            if l_rd is not None:
                for rd in l_rd:
                    rd.wait()
        contrib(rq_buf, ro_buf, R_HOPS, (my - R_HOPS) % N_DEV, first=False)

    bf = jnp.bfloat16
    i8 = jnp.int8
    return pl.pallas_call(
        body,
        out_shape=jax.ShapeDtypeStruct((B, SQL, D), jnp.float32),
        in_specs=[pl.BlockSpec(memory_space=pltpu.VMEM)] * 5,
        out_specs=pl.BlockSpec(memory_space=pltpu.VMEM),
        scratch_shapes=[
            pltpu.VMEM((B * SQL, D), bf),
            pltpu.VMEM((R_HOPS + 1, D, HB * DH), i8),
            pltpu.VMEM((R_HOPS + 1, HB * DH, D), i8),
            pltpu.VMEM((L_HOPS + 1, D, HB * DH), i8),
            pltpu.VMEM((L_HOPS + 1, HB * DH, D), i8),
            pltpu.SemaphoreType.DMA((R_HOPS,)),
            pltpu.SemaphoreType.DMA((R_HOPS,)),
            pltpu.SemaphoreType.DMA((R_HOPS,)),
            pltpu.SemaphoreType.DMA((R_HOPS,)),
            pltpu.SemaphoreType.DMA((L_HOPS,)),
            pltpu.SemaphoreType.DMA((L_HOPS,)),
            pltpu.SemaphoreType.DMA((L_HOPS,)),
            pltpu.SemaphoreType.DMA((L_HOPS,)),
        ],
        compiler_params=pltpu.CompilerParams(collective_id=0),
    )(x, Wq, K_r, V_r, Wo)


# device time: 33799 ns/iter; 1.5893x vs baseline; 1.0411x over previous
import jax
import jax.numpy as jnp
from jax import lax
from jax.experimental import pallas as pl
from jax.experimental.pallas import tpu as pltpu

N_DEV = 8
B = 2
SQL = 256
D = 512
HB = 4
DH = 64
SKV = 256

R_HOPS = 4
L_HOPS = 3

W_SIGMA = 0.02
QSCALE = 127.0 / (4.0 * W_SIGMA)


def _mm(a, b, out_dtype=jnp.float32):
    return lax.dot_general(
        a, b, (((1,), (0,)), ((), ())), preferred_element_type=out_dtype
    )


def kernel(x, Wq, K_ext, V_ext, Wo):
    K_r = jnp.transpose(K_ext, (0, 2, 3, 1)).astype(jnp.bfloat16)
    V_r = (jnp.transpose(V_ext, (0, 2, 1, 3)) * (1.0 / QSCALE)).astype(
        jnp.bfloat16
    )

    def body(
        x_ref, wq_ref, k_ref, v_ref, wo_ref, out_ref,
        xb, rq_buf, ro_buf, lq_buf, lo_buf,
        rq_s, rq_r, ro_s, ro_r, lq_s, lq_r, lo_s, lo_r,
    ):
        my = lax.axis_index("i")
        left = (my - 1) % N_DEV
        right = (my + 1) % N_DEV

        barrier_sem = pltpu.get_barrier_semaphore()
        for nbr in (left, right):
            pl.semaphore_signal(
                barrier_sem, inc=1, device_id=(nbr,),
                device_id_type=pl.DeviceIdType.MESH,
            )
        pl.semaphore_wait(barrier_sem, 2)

        xb[...] = (
            x_ref[...].reshape(B * SQL, D) * (0.125 / QSCALE)
        ).astype(jnp.bfloat16)
        wq8 = jnp.clip(
            jnp.round(wq_ref[...] * QSCALE), -127.0, 127.0
        ).astype(jnp.int8)
        wo8 = jnp.clip(
            jnp.round(wo_ref[...] * QSCALE), -127.0, 127.0
        ).astype(jnp.int8)
        rq_buf[0] = wq8
        ro_buf[0] = wo8
        lq_buf[0] = wq8
        lo_buf[0] = wo8

        qi = lax.broadcasted_iota(jnp.int32, (SQL, SKV), 0)
        kj = lax.broadcasted_iota(jnp.int32, (SQL, SKV), 1)
        qb = my * HB + qi // 64
        kb = kj // 64
        mask = (qb == kb) | (kb == 0) | ((qb + kb) % 3 == 0)

        def contrib(qbuf, obuf, slot, origin, first):
            wq_s = qbuf[slot].astype(jnp.bfloat16)
            wo_s = obuf[slot].astype(jnp.bfloat16)
            q16 = _mm(xb[...], wq_s).astype(jnp.bfloat16)
            parts = []
            for b in range(B):
                kblk = k_ref[b, pl.ds(origin * HB, HB)]
                vblk = v_ref[b, pl.ds(origin * HB, HB)]
                ctxs = []
                for h in range(HB):
                    qh = q16[b * SQL:(b + 1) * SQL, h * DH:(h + 1) * DH]
                    s = _mm(qh, kblk[h])
                    w = jnp.where(mask, jnp.exp(s), 0.0)
                    wsum = jnp.sum(w, axis=1, keepdims=True)
                    ctx = _mm(w.astype(jnp.bfloat16), vblk[h]) / wsum
                    ctxs.append(ctx.astype(jnp.bfloat16))
                parts.append(jnp.concatenate(ctxs, axis=1))
            ctx_all = jnp.concatenate(parts, axis=0)
            pall = _mm(ctx_all, wo_s).reshape(B, SQL, D)
            if first:
                out_ref[...] = pall
            else:
                out_ref[...] = out_ref[...] + pall

        def hop1(buf, s_sems, r_sems, idx, dst):
            rd = pltpu.make_async_remote_copy(
                src_ref=buf.at[idx], dst_ref=buf.at[idx + 1],
                send_sem=s_sems.at[idx], recv_sem=r_sems.at[idx],
                device_id=(dst,), device_id_type=pl.DeviceIdType.MESH,
            )
            rd.start()
            return rd

        for k in range(R_HOPS):
            rds = [hop1(rq_buf, rq_s, rq_r, k, right)]
            if k < R_HOPS - 1:
                rds.append(hop1(ro_buf, ro_s, ro_r, k, right))
            rds.append(hop1(lo_buf, lo_s, lo_r, k, left))
            if k < L_HOPS:
                rds.append(hop1(lq_buf, lq_s, lq_r, k, left))
            if k == 0:
                contrib(rq_buf, ro_buf, 0, my, first=True)
            else:
                contrib(rq_buf, ro_buf, k, (my - k) % N_DEV, first=False)
                contrib(lq_buf, lo_buf, k, (my + k) % N_DEV, first=False)
            for rd in rds:
                rd.wait()
        contrib(rq_buf, lo_buf, R_HOPS, (my - R_HOPS) % N_DEV, first=False)

    bf = jnp.bfloat16
    i8 = jnp.int8
    return pl.pallas_call(
        body,
        out_shape=jax.ShapeDtypeStruct((B, SQL, D), jnp.float32),
        in_specs=[pl.BlockSpec(memory_space=pltpu.VMEM)] * 5,
        out_specs=pl.BlockSpec(memory_space=pltpu.VMEM),
        scratch_shapes=[
            pltpu.VMEM((B * SQL, D), bf),
            pltpu.VMEM((R_HOPS + 1, D, HB * DH), i8),
            pltpu.VMEM((R_HOPS, HB * DH, D), i8),
            pltpu.VMEM((L_HOPS + 1, D, HB * DH), i8),
            pltpu.VMEM((R_HOPS + 1, HB * DH, D), i8),
            pltpu.SemaphoreType.DMA((R_HOPS,)),
            pltpu.SemaphoreType.DMA((R_HOPS,)),
            pltpu.SemaphoreType.DMA((R_HOPS - 1,)),
            pltpu.SemaphoreType.DMA((R_HOPS - 1,)),
            pltpu.SemaphoreType.DMA((L_HOPS,)),
            pltpu.SemaphoreType.DMA((L_HOPS,)),
            pltpu.SemaphoreType.DMA((R_HOPS,)),
            pltpu.SemaphoreType.DMA((R_HOPS,)),
        ],
        compiler_params=pltpu.CompilerParams(collective_id=0),
    )(x, Wq, K_r, V_r, Wo)


# device time: 31927 ns/iter; 1.6825x vs baseline; 1.0586x over previous
import jax
import jax.numpy as jnp
from jax import lax
from jax.experimental import pallas as pl
from jax.experimental.pallas import tpu as pltpu

N_DEV = 8
B = 2
SQL = 256
D = 512
HB = 4
DH = 64
SKV = 256

R_HOPS = 4
L_HOPS = 3

W_SIGMA = 0.02
QSCALE = 127.0 / (4.0 * W_SIGMA)


def _mm(a, b, out_dtype=jnp.float32):
    return lax.dot_general(
        a, b, (((1,), (0,)), ((), ())), preferred_element_type=out_dtype
    )


def kernel(x, Wq, K_ext, V_ext, Wo):
    K_r = jnp.transpose(K_ext, (0, 2, 3, 1)).astype(jnp.bfloat16)
    V_r = (jnp.transpose(V_ext, (0, 2, 1, 3)) * (1.0 / QSCALE)).astype(
        jnp.bfloat16
    )

    def body(
        x_ref, wq_ref, k_ref, v_ref, wo_ref, out_ref,
        xb, rq_buf, ro_buf, lq_buf, lo_buf,
        rq_s, rq_r, ro_s, ro_r, lq_s, lq_r, lo_s, lo_r,
    ):
        my = lax.axis_index("i")

        def perm(p):
            return jnp.where(p < 4, p, 11 - p)

        pos = perm(my)
        left = perm((pos - 1) % N_DEV)
        right = perm((pos + 1) % N_DEV)

        barrier_sem = pltpu.get_barrier_semaphore()
        for nbr in (left, right):
            pl.semaphore_signal(
                barrier_sem, inc=1, device_id=(nbr,),
                device_id_type=pl.DeviceIdType.MESH,
            )
        pl.semaphore_wait(barrier_sem, 2)

        xb[...] = (
            x_ref[...].reshape(B * SQL, D) * (0.125 / QSCALE)
        ).astype(jnp.bfloat16)
        wq8 = jnp.clip(
            jnp.round(wq_ref[...] * QSCALE), -127.0, 127.0
        ).astype(jnp.int8)
        wo8 = jnp.clip(
            jnp.round(wo_ref[...] * QSCALE), -127.0, 127.0
        ).astype(jnp.int8)
        rq_buf[0] = wq8
        ro_buf[0] = wo8
        lq_buf[0] = wq8
        lo_buf[0] = wo8

        qi = lax.broadcasted_iota(jnp.int32, (SQL, SKV), 0)
        kj = lax.broadcasted_iota(jnp.int32, (SQL, SKV), 1)
        qb = my * HB + qi // 64
        kb = kj // 64
        mask = (qb == kb) | (kb == 0) | ((qb + kb) % 3 == 0)

        def contrib(qbuf, obuf, slot, origin, first):
            wq_s = qbuf[slot].astype(jnp.bfloat16)
            wo_s = obuf[slot].astype(jnp.bfloat16)
            q16 = _mm(xb[...], wq_s).astype(jnp.bfloat16)
            parts = []
            for b in range(B):
                kblk = k_ref[b, pl.ds(origin * HB, HB)]
                vblk = v_ref[b, pl.ds(origin * HB, HB)]
                ctxs = []
                for h in range(HB):
                    qh = q16[b * SQL:(b + 1) * SQL, h * DH:(h + 1) * DH]
                    s = _mm(qh, kblk[h])
                    w = jnp.where(mask, jnp.exp(s), 0.0)
                    wsum = jnp.sum(w, axis=1, keepdims=True)
                    ctx = _mm(w.astype(jnp.bfloat16), vblk[h]) / wsum
                    ctxs.append(ctx.astype(jnp.bfloat16))
                parts.append(jnp.concatenate(ctxs, axis=1))
            ctx_all = jnp.concatenate(parts, axis=0)
            pall = _mm(ctx_all, wo_s).reshape(B, SQL, D)
            if first:
                out_ref[...] = pall
            else:
                out_ref[...] = out_ref[...] + pall

        def hop1(buf, s_sems, r_sems, idx, dst):
            rd = pltpu.make_async_remote_copy(
                src_ref=buf.at[idx], dst_ref=buf.at[idx + 1],
                send_sem=s_sems.at[idx], recv_sem=r_sems.at[idx],
                device_id=(dst,), device_id_type=pl.DeviceIdType.MESH,
            )
            rd.start()
            return rd

        for k in range(R_HOPS):
            rds = [hop1(rq_buf, rq_s, rq_r, k, right)]
            if k < R_HOPS - 1:
                rds.append(hop1(ro_buf, ro_s, ro_r, k, right))
            rds.append(hop1(lo_buf, lo_s, lo_r, k, left))
            if k < L_HOPS:
                rds.append(hop1(lq_buf, lq_s, lq_r, k, left))
            if k == 0:
                contrib(rq_buf, ro_buf, 0, my, first=True)
            else:
                contrib(rq_buf, ro_buf, k, perm((pos - k) % N_DEV), False)
                contrib(lq_buf, lo_buf, k, perm((pos + k) % N_DEV), False)
            for rd in rds:
                rd.wait()
        contrib(rq_buf, lo_buf, R_HOPS, perm((pos - R_HOPS) % N_DEV), False)

    bf = jnp.bfloat16
    i8 = jnp.int8
    return pl.pallas_call(
        body,
        out_shape=jax.ShapeDtypeStruct((B, SQL, D), jnp.float32),
        in_specs=[pl.BlockSpec(memory_space=pltpu.VMEM)] * 5,
        out_specs=pl.BlockSpec(memory_space=pltpu.VMEM),
        scratch_shapes=[
            pltpu.VMEM((B * SQL, D), bf),
            pltpu.VMEM((R_HOPS + 1, D, HB * DH), i8),
            pltpu.VMEM((R_HOPS, HB * DH, D), i8),
            pltpu.VMEM((L_HOPS + 1, D, HB * DH), i8),
            pltpu.VMEM((R_HOPS + 1, HB * DH, D), i8),
            pltpu.SemaphoreType.DMA((R_HOPS,)),
            pltpu.SemaphoreType.DMA((R_HOPS,)),
            pltpu.SemaphoreType.DMA((R_HOPS - 1,)),
            pltpu.SemaphoreType.DMA((R_HOPS - 1,)),
            pltpu.SemaphoreType.DMA((L_HOPS,)),
            pltpu.SemaphoreType.DMA((L_HOPS,)),
            pltpu.SemaphoreType.DMA((R_HOPS,)),
            pltpu.SemaphoreType.DMA((R_HOPS,)),
        ],
        compiler_params=pltpu.CompilerParams(collective_id=0),
    )(x, Wq, K_r, V_r, Wo)
